# baseline (speedup 1.0000x reference)
import functools

import jax
import jax.numpy as jnp
from jax import lax
from jax.experimental import pallas as pl
from jax.experimental.pallas import tpu as pltpu
from jax.experimental.pallas import tpu_sc as plsc

_U = 10000
_V = 10000
_N = _U + _V
_F = 512
_R = 5
_E = 30000
_ED = _R * _E

_FC = 4
_FW = 128

_NC = 2
_NS = 16
_B = 125
_BP = 128
_NB = 76
_EDP = _NS * _NB * _B
_VP = 10240
_DUMMY = _VP
_ACC_ROWS = _VP + 16
_RPT = _VP // _NS


def _mm_body(x_ref, w_ref, o_ref):
    o_ref[...] = jnp.dot(x_ref[...], w_ref[0],
                         preferred_element_type=jnp.float32)


_MBLK = 2000


def _project(x, weight):
    grid = (_N // _MBLK, _R, _FC)
    return pl.pallas_call(
        _mm_body,
        grid=grid,
        in_specs=[
            pl.BlockSpec((_MBLK, _F), lambda m, i, f: (m, 0)),
            pl.BlockSpec((1, _F, _FW), lambda m, i, f: (i, 0, f)),
        ],
        out_specs=pl.BlockSpec(
            (_MBLK, _FW),
            lambda m, i, f: (f * (_R * _N // _MBLK) + i * (_N // _MBLK) + m, 0)),
        out_shape=jax.ShapeDtypeStruct((_FC * _R * _N, _FW), jnp.float32),
    )(x, weight)


def _sc_agg(table, srcs, dsts, zeros):
    mesh = plsc.VectorSubcoreMesh(core_axis_name="c", subcore_axis_name="s")

    @functools.partial(
        pl.kernel,
        mesh=mesh,
        out_type=jax.ShapeDtypeStruct((_NC, _FC, _VP, _FW), jnp.float32),
        scratch_types=[
            pltpu.VMEM((_NB // 2, _BP), jnp.int32),
            pltpu.VMEM((_NB // 2, _BP), jnp.int32),
            pltpu.VMEM((_BP, _FW), jnp.float32),
            pltpu.VMEM((_BP, _FW), jnp.float32),
            pltpu.VMEM_SHARED((_ACC_ROWS, _FW), jnp.float32),
            pltpu.SemaphoreType.DMA,
            pltpu.SemaphoreType.DMA,
            pltpu.SemaphoreType.DMA,
            pltpu.SemaphoreType.DMA,
        ],
    )
    def body(table_h, srcs_h, dsts_h, zeros_h, out_h,
             src_v, dst_v, rows_a, rows_b, acc,
             sem_ga, sem_gb, sem_sa, sem_sb):
        c = lax.axis_index("c")
        s = lax.axis_index("s")
        row0 = s * _RPT
        for fc in range(_FC):
            pltpu.sync_copy(zeros_h, acc.at[pl.ds(row0, _RPT)])

            @pl.when(s == _NS - 1)
            def _():
                pltpu.sync_copy(zeros_h.at[pl.ds(0, 16)],
                                acc.at[pl.ds(_VP, 16)])

            plsc.subcore_barrier()

            def step(h, _):
                b0 = 2 * h
                b1 = b0 + 1
                ga = pltpu.async_copy(table_h.at[src_v.at[b0]], rows_a,
                                      sem_ga)
                gb = pltpu.async_copy(table_h.at[src_v.at[b1]], rows_b,
                                      sem_gb)
                ga.wait()
                sa = pltpu.async_copy(rows_a, acc.at[dst_v.at[b0]], sem_sa,
                                      add=True)
                gb.wait()
                sa.wait()
                sb = pltpu.async_copy(rows_b, acc.at[dst_v.at[b1]], sem_sb,
                                      add=True)
                sb.wait()
                return _

            for half in range(2):
                pltpu.sync_copy(srcs_h.at[fc, c, s, half], src_v)
                pltpu.sync_copy(dsts_h.at[c, s, half], dst_v)
                lax.fori_loop(0, _NB // 4, step, None)
            plsc.subcore_barrier()
            pltpu.sync_copy(acc.at[pl.ds(row0, _RPT)],
                            out_h.at[c, fc, pl.ds(row0, _RPT)])
            plsc.subcore_barrier()

    return body(table, srcs, dsts, zeros)


def _scale_body(a0_ref, a1_ref, cu_ref, cv_ref, hu_ref, hv_ref):
    cv = cv_ref[...]
    cu = cu_ref[...]
    hv_ref[...] = a0_ref[0, 0] * (cv * cv)
    hu_ref[...] = a1_ref[0, 0] * (cu * cu)


def _scale(acc, ci_u, ci_v):
    grid = (_V // _MBLK, _FC)
    return pl.pallas_call(
        _scale_body,
        grid=grid,
        in_specs=[
            pl.BlockSpec((1, 1, _MBLK, _FW), lambda m, f: (0, f, m, 0)),
            pl.BlockSpec((1, 1, _MBLK, _FW), lambda m, f: (1, f, m, 0)),
            pl.BlockSpec((_MBLK, 1), lambda m, f: (m, 0)),
            pl.BlockSpec((_MBLK, 1), lambda m, f: (m, 0)),
        ],
        out_specs=[
            pl.BlockSpec((_MBLK, _FW), lambda m, f: (m, f)),
            pl.BlockSpec((_MBLK, _FW), lambda m, f: (m, f)),
        ],
        out_shape=[
            jax.ShapeDtypeStruct((_U, _F), jnp.float32),
            jax.ShapeDtypeStruct((_V, _F), jnp.float32),
        ],
    )(acc, acc, ci_u, ci_v)


def kernel(x_u, x_v, ci_u, ci_v, edge_u, edge_v, weight):
    x = jnp.concatenate([x_u, x_v], axis=0)
    table = _project(x, weight)

    def _tile_split(a, pad_val):
        flat = a.reshape(-1)
        flat = jnp.pad(flat, (0, _EDP - _ED), constant_values=pad_val)
        return flat.reshape(_NS, 2, _NB // 2, _B)

    roff = (jnp.arange(_R, dtype=jnp.int32) * _N)[:, None]
    src_v = _tile_split(edge_u + roff, 0)
    src_u = _tile_split(edge_v + roff + _U, 0)
    src = jnp.stack([src_v, src_u])
    src = jnp.pad(src, ((0, 0),) * 4 + ((0, _BP - _B),))
    fcoff = (jnp.arange(_FC, dtype=jnp.int32) * (_R * _N)
             ).reshape(_FC, 1, 1, 1, 1, 1)
    srcs = src[None] + fcoff

    dst = jnp.stack([_tile_split(edge_v, _DUMMY),
                     _tile_split(edge_u, _DUMMY)])
    dsts = jnp.pad(dst, ((0, 0),) * 4 + ((0, _BP - _B),),
                   constant_values=_DUMMY)

    zeros = jnp.zeros((_RPT, _FW), jnp.float32)
    acc = _sc_agg(table, srcs, dsts, zeros)

    h_u, h_v = _scale(acc, ci_u[:, None], ci_v[:, None])
    return (h_u, h_v)


# speedup vs baseline: 1.0003x; 1.0003x over previous
import functools

import jax
import jax.numpy as jnp
from jax import lax
from jax.experimental import pallas as pl
from jax.experimental.pallas import tpu as pltpu
from jax.experimental.pallas import tpu_sc as plsc

_U = 10000
_V = 10000
_N = _U + _V
_F = 512
_R = 5
_E = 30000
_ED = _R * _E

_FC = 4
_FW = 128

_NC = 2
_NS = 16
_B = 125
_BP = 128
_NB = 76
_EDP = _NS * _NB * _B
_VP = 10240
_DUMMY = _VP
_ACC_ROWS = _VP + 16
_RPT = _VP // _NS


def _mm_body(x_ref, w_ref, o_ref):
    o_ref[...] = jnp.dot(x_ref[...], w_ref[0],
                         preferred_element_type=jnp.float32)


_MBLK = 2000


def _project(x, weight):
    grid = (_N // _MBLK, _R, _FC)
    return pl.pallas_call(
        _mm_body,
        grid=grid,
        in_specs=[
            pl.BlockSpec((_MBLK, _F), lambda m, i, f: (m, 0)),
            pl.BlockSpec((1, _F, _FW), lambda m, i, f: (i, 0, f)),
        ],
        out_specs=pl.BlockSpec(
            (_MBLK, _FW),
            lambda m, i, f: (f * (_R * _N // _MBLK) + i * (_N // _MBLK) + m, 0)),
        out_shape=jax.ShapeDtypeStruct((_FC * _R * _N, _FW), jnp.float32),
    )(x, weight)


def _sc_agg(table, srcs, dsts, zeros):
    mesh = plsc.VectorSubcoreMesh(core_axis_name="c", subcore_axis_name="s")

    @functools.partial(
        pl.kernel,
        mesh=mesh,
        out_type=jax.ShapeDtypeStruct((_NC, _FC, _VP, _FW), jnp.float32),
        scratch_types=[
            pltpu.VMEM((_NB // 2, _BP), jnp.int32),
            pltpu.VMEM((_NB // 2, _BP), jnp.int32),
            pltpu.VMEM((_BP, _FW), jnp.float32),
            pltpu.VMEM((_BP, _FW), jnp.float32),
            pltpu.VMEM_SHARED((_ACC_ROWS, _FW), jnp.float32),
            pltpu.SemaphoreType.DMA,
            pltpu.SemaphoreType.DMA,
            pltpu.SemaphoreType.DMA,
            pltpu.SemaphoreType.DMA,
        ],
    )
    def body(table_h, srcs_h, dsts_h, zeros_h, out_h,
             src_v, dst_v, rows_a, rows_b, acc,
             sem_ga, sem_gb, sem_sa, sem_sb):
        c = lax.axis_index("c")
        s = lax.axis_index("s")
        row0 = s * _RPT
        for fc in range(_FC):
            pltpu.sync_copy(zeros_h, acc.at[pl.ds(row0, _RPT)])

            @pl.when(s == _NS - 1)
            def _():
                pltpu.sync_copy(zeros_h.at[pl.ds(0, 16)],
                                acc.at[pl.ds(_VP, 16)])

            plsc.subcore_barrier()

            def step(h, _):
                b0 = 2 * h
                b1 = b0 + 1
                ga = pltpu.async_copy(table_h.at[src_v.at[b0]], rows_a,
                                      sem_ga)
                gb = pltpu.async_copy(table_h.at[src_v.at[b1]], rows_b,
                                      sem_gb)
                ga.wait()
                pltpu.sync_copy(rows_a, acc.at[dst_v.at[b0]], add=True)
                gb.wait()
                pltpu.sync_copy(rows_b, acc.at[dst_v.at[b1]], add=True)
                return _

            for half in range(2):
                pltpu.sync_copy(srcs_h.at[fc, c, s, half], src_v)
                pltpu.sync_copy(dsts_h.at[c, s, half], dst_v)
                lax.fori_loop(0, _NB // 4, step, None)
            plsc.subcore_barrier()
            pltpu.sync_copy(acc.at[pl.ds(row0, _RPT)],
                            out_h.at[c, fc, pl.ds(row0, _RPT)])
            plsc.subcore_barrier()

    return body(table, srcs, dsts, zeros)


def _scale_body(a0_ref, a1_ref, cu_ref, cv_ref, hu_ref, hv_ref):
    cv = cv_ref[...]
    cu = cu_ref[...]
    hv_ref[...] = a0_ref[0, 0] * (cv * cv)
    hu_ref[...] = a1_ref[0, 0] * (cu * cu)


def _scale(acc, ci_u, ci_v):
    grid = (_V // _MBLK, _FC)
    return pl.pallas_call(
        _scale_body,
        grid=grid,
        in_specs=[
            pl.BlockSpec((1, 1, _MBLK, _FW), lambda m, f: (0, f, m, 0)),
            pl.BlockSpec((1, 1, _MBLK, _FW), lambda m, f: (1, f, m, 0)),
            pl.BlockSpec((_MBLK, 1), lambda m, f: (m, 0)),
            pl.BlockSpec((_MBLK, 1), lambda m, f: (m, 0)),
        ],
        out_specs=[
            pl.BlockSpec((_MBLK, _FW), lambda m, f: (m, f)),
            pl.BlockSpec((_MBLK, _FW), lambda m, f: (m, f)),
        ],
        out_shape=[
            jax.ShapeDtypeStruct((_U, _F), jnp.float32),
            jax.ShapeDtypeStruct((_V, _F), jnp.float32),
        ],
    )(acc, acc, ci_u, ci_v)


def kernel(x_u, x_v, ci_u, ci_v, edge_u, edge_v, weight):
    x = jnp.concatenate([x_u, x_v], axis=0)
    table = _project(x, weight)

    def _tile_split(a, pad_val):
        flat = a.reshape(-1)
        flat = jnp.pad(flat, (0, _EDP - _ED), constant_values=pad_val)
        return flat.reshape(_NS, 2, _NB // 2, _B)

    roff = (jnp.arange(_R, dtype=jnp.int32) * _N)[:, None]
    src_v = _tile_split(edge_u + roff, 0)
    src_u = _tile_split(edge_v + roff + _U, 0)
    src = jnp.stack([src_v, src_u])
    src = jnp.pad(src, ((0, 0),) * 4 + ((0, _BP - _B),))
    fcoff = (jnp.arange(_FC, dtype=jnp.int32) * (_R * _N)
             ).reshape(_FC, 1, 1, 1, 1, 1)
    srcs = src[None] + fcoff

    dst = jnp.stack([_tile_split(edge_v, _DUMMY),
                     _tile_split(edge_u, _DUMMY)])
    dsts = jnp.pad(dst, ((0, 0),) * 4 + ((0, _BP - _B),),
                   constant_values=_DUMMY)

    zeros = jnp.zeros((_RPT, _FW), jnp.float32)
    acc = _sc_agg(table, srcs, dsts, zeros)

    h_u, h_v = _scale(acc, ci_u[:, None], ci_v[:, None])
    return (h_u, h_v)
